# Initial kernel scaffold; baseline (speedup 1.0000x reference)
import functools

import jax
import jax.numpy as jnp
from jax import lax
from jax.experimental import pallas as pl
from jax.experimental.pallas import tpu as pltpu
from jax.experimental.pallas import tpu_sc as plsc

N = 10000
E = 320000
D_HID = 128
D_OUT = 40
D_OUTP = 64
EPS = 1e-5

NC = 2
NS = 16
NW = NC * NS
CHUNK = 128
CH = 80
EPT = CH * CHUNK
ETOT = NW * EPT
NPAD = 10112
SR = 79
NB = 16
BR = NPAD // NB


def _sc_mesh():
    return plsc.VectorSubcoreMesh(
        core_axis_name="c", subcore_axis_name="s", num_cores=NC, num_subcores=NS
    )


def _sc_degree(dstp, zeros_hbm):
    @functools.partial(
        pl.kernel,
        out_type=jax.ShapeDtypeStruct((NW, SR, 128), jnp.float32),
        mesh=_sc_mesh(),
        scratch_types=[
            pltpu.VMEM((CH, CHUNK), jnp.int32),
            pltpu.VMEM((SR, 128), jnp.float32),
        ],
    )
    def k(dst_hbm, zero_hbm, out_hbm, dst_v, hist_v):
        c = lax.axis_index("c")
        s = lax.axis_index("s")
        wid = c * NS + s
        pltpu.sync_copy(dst_hbm.at[wid], dst_v)
        pltpu.sync_copy(zero_hbm, hist_v)

        ones = jnp.full((16,), 1.0, jnp.float32)

        def body(j, _):
            for kk in range(CHUNK // 16):
                idx = dst_v[j, pl.ds(kk * 16, 16)]
                row = lax.shift_right_logical(idx, 7)
                col = lax.bitwise_and(idx, 127)
                plsc.addupdate_scatter(hist_v, [row, col], ones)
            return 0

        lax.fori_loop(0, CH, body, 0)
        pltpu.sync_copy(hist_v, out_hbm.at[wid])

    return k(dstp, zeros_hbm)


def _tc_degsum(degp):
    def body(p_ref, o_ref):
        o_ref[...] = jnp.sum(p_ref[...], axis=0) + 1.0

    return pl.pallas_call(
        body,
        out_shape=jax.ShapeDtypeStruct((SR, 128), jnp.float32),
    )(degp)


def _sc_agg(hs, srcp, dstp, zeros_hbm, d):
    @functools.partial(
        pl.kernel,
        out_type=jax.ShapeDtypeStruct((NC, NPAD, d), jnp.float32),
        mesh=_sc_mesh(),
        scratch_types=[
            pltpu.VMEM((CH, CHUNK), jnp.int32),
            pltpu.VMEM((CH, CHUNK), jnp.int32),
            pltpu.VMEM((CHUNK, d), jnp.float32),
            pltpu.VMEM((CHUNK, d), jnp.float32),
            pltpu.VMEM((SR, d), jnp.float32),
            pltpu.VMEM_SHARED((NPAD, d), jnp.float32),
            pltpu.SemaphoreType.DMA,
            pltpu.SemaphoreType.DMA,
        ],
    )
    def k(hs_hbm, src_hbm, dst_hbm, zero_hbm, out_hbm,
          src_v, dst_v, rows_a, rows_b, bounce_v, acc_s, sem_a, sem_b):
        c = lax.axis_index("c")
        s = lax.axis_index("s")
        wid = c * NS + s
        pltpu.sync_copy(src_hbm.at[wid], src_v)
        pltpu.sync_copy(dst_hbm.at[wid], dst_v)
        pltpu.sync_copy(zero_hbm, bounce_v)
        for q in range(8):
            row0 = (s * 8 + q) * SR
            pltpu.sync_copy(bounce_v, acc_s.at[pl.ds(row0, SR)])
        plsc.subcore_barrier()

        pltpu.async_copy(hs_hbm.at[src_v.at[0]], rows_a, sem_a)

        def body(jj, _):
            j = jj * 2
            pltpu.make_async_copy(hs_hbm.at[src_v.at[j]], rows_a, sem_a).wait()
            pltpu.async_copy(hs_hbm.at[src_v.at[j + 1]], rows_b, sem_b)
            pltpu.sync_copy(rows_a, acc_s.at[dst_v.at[j]], add=True)
            pltpu.make_async_copy(hs_hbm.at[src_v.at[j + 1]], rows_b, sem_b).wait()

            @pl.when(jj < CH // 2 - 1)
            def _():
                pltpu.async_copy(hs_hbm.at[src_v.at[j + 2]], rows_a, sem_a)

            pltpu.sync_copy(rows_b, acc_s.at[dst_v.at[j + 1]], add=True)
            return 0

        lax.fori_loop(0, CH // 2, body, 0)
        plsc.subcore_barrier()

        for q in range(8):
            row0 = (s * 8 + q) * SR
            pltpu.sync_copy(acc_s.at[pl.ds(row0, SR)], bounce_v)
            pltpu.sync_copy(bounce_v, out_hbm.at[c, pl.ds(row0, SR)])

    return k(hs, srcp, dstp, zeros_hbm)


def _tc_matmul_scale(xp, W, deg1):
    kdim, d = W.shape

    def body(x_ref, w_ref, deg_ref, o_ref):
        h = jnp.dot(x_ref[...], w_ref[...], preferred_element_type=jnp.float32)
        o_ref[...] = h * lax.rsqrt(deg_ref[...])

    return pl.pallas_call(
        body,
        grid=(NB,),
        in_specs=[
            pl.BlockSpec((BR, kdim), lambda i: (i, 0)),
            pl.BlockSpec((kdim, d), lambda i: (0, 0)),
            pl.BlockSpec((BR, 1), lambda i: (i, 0)),
        ],
        out_specs=pl.BlockSpec((BR, d), lambda i: (i, 0)),
        out_shape=jax.ShapeDtypeStruct((NPAD, d), jnp.float32),
    )(xp, W, deg1)


def _tc_bn_relu_matmul_scale(z, scale, shift, W, deg1):
    kdim, d = W.shape

    def body(z_ref, sc_ref, sh_ref, w_ref, deg_ref, o_ref):
        a = jax.nn.relu(z_ref[...] * sc_ref[...] + sh_ref[...])
        h = jnp.dot(a, w_ref[...], preferred_element_type=jnp.float32)
        o_ref[...] = h * lax.rsqrt(deg_ref[...])

    return pl.pallas_call(
        body,
        grid=(NB,),
        in_specs=[
            pl.BlockSpec((BR, kdim), lambda i: (i, 0)),
            pl.BlockSpec((1, kdim), lambda i: (0, 0)),
            pl.BlockSpec((1, kdim), lambda i: (0, 0)),
            pl.BlockSpec((kdim, d), lambda i: (0, 0)),
            pl.BlockSpec((BR, 1), lambda i: (i, 0)),
        ],
        out_specs=pl.BlockSpec((BR, d), lambda i: (i, 0)),
        out_shape=jax.ShapeDtypeStruct((NPAD, d), jnp.float32),
    )(z, scale, shift, W, deg1)


def _tc_z_stats(p, hs, deg1, b):
    d = hs.shape[1]

    def body(p_ref, hs_ref, deg_ref, b_ref, z_ref, st_ref, acc_ref):
        i = pl.program_id(0)
        z = lax.rsqrt(deg_ref[...]) * (p_ref[0] + p_ref[1] + hs_ref[...]) + b_ref[...]
        rows = i * BR + lax.broadcasted_iota(jnp.int32, (BR, d), 0)
        zm = jnp.where(rows < N, z, 0.0)
        z_ref[...] = zm

        @pl.when(i == 0)
        def _():
            acc_ref[...] = jnp.zeros_like(acc_ref)

        sums = jnp.concatenate(
            [jnp.sum(zm, axis=0, keepdims=True),
             jnp.sum(zm * zm, axis=0, keepdims=True)], axis=0)
        acc_ref[...] += sums

        @pl.when(i == NB - 1)
        def _():
            st_ref[...] = acc_ref[...]

    return pl.pallas_call(
        body,
        grid=(NB,),
        in_specs=[
            pl.BlockSpec((NC, BR, d), lambda i: (0, i, 0)),
            pl.BlockSpec((BR, d), lambda i: (i, 0)),
            pl.BlockSpec((BR, 1), lambda i: (i, 0)),
            pl.BlockSpec((1, d), lambda i: (0, 0)),
        ],
        out_specs=[
            pl.BlockSpec((BR, d), lambda i: (i, 0)),
            pl.BlockSpec((2, d), lambda i: (0, 0)),
        ],
        out_shape=[
            jax.ShapeDtypeStruct((NPAD, d), jnp.float32),
            jax.ShapeDtypeStruct((2, d), jnp.float32),
        ],
        scratch_shapes=[pltpu.VMEM((2, d), jnp.float32)],
        compiler_params=pltpu.CompilerParams(dimension_semantics=("arbitrary",)),
    )(p, hs, deg1, b)


def _tc_final(p, hs, deg1, b):
    d = D_OUTP
    rb = N // NB

    def body(p_ref, hs_ref, deg_ref, b_ref, o_ref):
        z = lax.rsqrt(deg_ref[...]) * (p_ref[0] + p_ref[1] + hs_ref[...]) + b_ref[...]
        cols = lax.broadcasted_iota(jnp.int32, (rb, d), 1)
        zm = jnp.where(cols < D_OUT, z, -jnp.inf)
        m = jnp.max(zm, axis=1, keepdims=True)
        lse = jnp.log(jnp.sum(jnp.exp(zm - m), axis=1, keepdims=True))
        o_ref[...] = (z - m - lse)[:, :D_OUT]

    return pl.pallas_call(
        body,
        grid=(NB,),
        in_specs=[
            pl.BlockSpec((NC, rb, d), lambda i: (0, i, 0)),
            pl.BlockSpec((rb, d), lambda i: (i, 0)),
            pl.BlockSpec((rb, 1), lambda i: (i, 0)),
            pl.BlockSpec((1, d), lambda i: (0, 0)),
        ],
        out_specs=pl.BlockSpec((rb, D_OUT), lambda i: (i, 0)),
        out_shape=jax.ShapeDtypeStruct((N, D_OUT), jnp.float32),
    )(p, hs, deg1, b)


def kernel(x, edge_index, W1, b1, gamma1, beta1, W2, b2, gamma2, beta2, W3, b3):
    src = edge_index[0]
    dst = edge_index[1]
    padlen = ETOT - E
    fill = jnp.full((padlen,), N, jnp.int32)
    srcp = jnp.concatenate([src, fill]).reshape(NW, CH, CHUNK)
    dstp = jnp.concatenate([dst, fill]).reshape(NW, CH, CHUNK)
    zeros128 = jnp.zeros((SR, 128), jnp.float32)
    zeros64 = jnp.zeros((SR, D_OUTP), jnp.float32)

    degp = _sc_degree(dstp, zeros128)
    deg1 = _tc_degsum(degp).reshape(NPAD, 1)

    xp = jnp.pad(x, ((0, NPAD - N), (0, 0)))

    hs1 = _tc_matmul_scale(xp, W1, deg1)
    p1 = _sc_agg(hs1, srcp, dstp, zeros128, D_HID)
    z1, st1 = _tc_z_stats(p1, hs1, deg1, b1.reshape(1, -1))
    mean1 = st1[0] / N
    var1 = st1[1] / N - mean1 * mean1
    isd1 = gamma1 * lax.rsqrt(var1 + EPS)
    sc1 = isd1.reshape(1, -1)
    sh1 = (beta1 - mean1 * isd1).reshape(1, -1)

    hs2 = _tc_bn_relu_matmul_scale(z1, sc1, sh1, W2, deg1)
    p2 = _sc_agg(hs2, srcp, dstp, zeros128, D_HID)
    z2, st2 = _tc_z_stats(p2, hs2, deg1, b2.reshape(1, -1))
    mean2 = st2[0] / N
    var2 = st2[1] / N - mean2 * mean2
    isd2 = gamma2 * lax.rsqrt(var2 + EPS)
    sc2 = isd2.reshape(1, -1)
    sh2 = (beta2 - mean2 * isd2).reshape(1, -1)

    W3p = jnp.pad(W3, ((0, 0), (0, D_OUTP - D_OUT)))
    b3p = jnp.pad(b3, (0, D_OUTP - D_OUT)).reshape(1, -1)
    hs3 = _tc_bn_relu_matmul_scale(z2, sc2, sh2, W3p, deg1)
    p3 = _sc_agg(hs3, srcp, dstp, zeros64, D_OUTP)
    return _tc_final(p3, hs3, deg1, b3p)


# speedup vs baseline: 8.1135x; 8.1135x over previous
import functools

import jax
import jax.numpy as jnp
from jax import lax
from jax.experimental import pallas as pl
from jax.experimental.pallas import tpu as pltpu
from jax.experimental.pallas import tpu_sc as plsc

N = 10000
E = 320000
D_HID = 128
D_OUT = 40
D_OUTP = 64
EPS = 1e-5

NC = 2
NS = 16
NW = NC * NS
CHUNK = 128
CH = 80
EPT = CH * CHUNK
ETOT = NW * EPT
NPAD = 10112
SR = 79
DH = 64
NB = 16
BR = NPAD // NB
STRIPE = NPAD // NS
ZCH = 152
ZCHUNKS = ((0, 152), (152, 152), (304, 152), (456, 152), (608, 24))


def _sc_mesh():
    return plsc.VectorSubcoreMesh(
        core_axis_name="c", subcore_axis_name="s", num_cores=NC, num_subcores=NS
    )


def _sc_degree(dstp, zeros_hbm):
    @functools.partial(
        pl.kernel,
        out_type=jax.ShapeDtypeStruct((NW, NPAD), jnp.float32),
        mesh=_sc_mesh(),
        scratch_types=[
            pltpu.VMEM((CH, CHUNK), jnp.int32),
            pltpu.VMEM((NPAD,), jnp.float32),
        ],
        compiler_params=pltpu.CompilerParams(needs_layout_passes=False),
    )
    def k(dst_hbm, zero_hbm, out_hbm, dst_v, hist_v):
        c = lax.axis_index("c")
        s = lax.axis_index("s")
        wid = c * NS + s
        pltpu.sync_copy(dst_hbm.at[wid], dst_v)
        pltpu.sync_copy(zero_hbm, hist_v)

        ones = jnp.full((16,), 1.0, jnp.float32)

        def body(j, _):
            for kk in range(CHUNK // 16):
                idx = dst_v[j, pl.ds(kk * 16, 16)]
                plsc.addupdate_scatter(hist_v, [idx], ones)
            return 0

        lax.fori_loop(0, CH, body, 0)
        pltpu.sync_copy(hist_v, out_hbm.at[wid])

    return k(dstp, zeros_hbm)


def _tc_degsum(degp):
    def body(p_ref, o_ref):
        o_ref[...] = jnp.sum(p_ref[...], axis=0, keepdims=True) + 1.0

    return pl.pallas_call(
        body,
        out_shape=jax.ShapeDtypeStruct((1, NPAD), jnp.float32),
    )(degp)


def _sc_agg(hs_halves, srcp, dstp, zeros_hbm):
    H = len(hs_halves)

    @functools.partial(
        pl.kernel,
        out_type=jax.ShapeDtypeStruct((NC * H, NPAD, DH), jnp.float32),
        mesh=_sc_mesh(),
        scratch_types=[
            pltpu.VMEM((CH, CHUNK), jnp.int32),
            pltpu.VMEM((CH, CHUNK), jnp.int32),
            pltpu.VMEM((CHUNK, DH), jnp.float32),
            pltpu.VMEM((CHUNK, DH), jnp.float32),
            pltpu.VMEM((ZCH, DH), jnp.float32),
            pltpu.VMEM_SHARED((NPAD, DH), jnp.float32),
            pltpu.SemaphoreType.DMA,
            pltpu.SemaphoreType.DMA,
        ],
        compiler_params=pltpu.CompilerParams(use_tc_tiling_on_sc=False),
    )
    def k(*refs):
        hs_hbms = refs[:H]
        (src_hbm, dst_hbm, zero_hbm, out_hbm,
         src_v, dst_v, rows_a, rows_b, bounce_v, acc_s, sem_a, sem_b) = refs[H:]
        c = lax.axis_index("c")
        s = lax.axis_index("s")
        wid = c * NS + s
        pltpu.sync_copy(src_hbm.at[wid], src_v)
        pltpu.sync_copy(dst_hbm.at[wid], dst_v)
        base = s * STRIPE

        for h in range(H):
            hs_hbm = hs_hbms[h]
            pltpu.sync_copy(zero_hbm, bounce_v)
            for off, ln in ZCHUNKS:
                pltpu.sync_copy(bounce_v.at[pl.ds(0, ln)],
                                acc_s.at[pl.ds(base + off, ln)])
            plsc.subcore_barrier()

            pltpu.async_copy(hs_hbm.at[src_v.at[0]], rows_a, sem_a)

            def body(jj, _, hs_hbm=hs_hbm):
                j = jj * 2
                pltpu.make_async_copy(hs_hbm.at[src_v.at[j]], rows_a, sem_a).wait()
                pltpu.async_copy(hs_hbm.at[src_v.at[j + 1]], rows_b, sem_b)
                pltpu.sync_copy(rows_a, acc_s.at[dst_v.at[j]], add=True)
                pltpu.make_async_copy(hs_hbm.at[src_v.at[j + 1]], rows_b, sem_b).wait()

                @pl.when(jj < CH // 2 - 1)
                def _():
                    pltpu.async_copy(hs_hbm.at[src_v.at[j + 2]], rows_a, sem_a)

                pltpu.sync_copy(rows_b, acc_s.at[dst_v.at[j + 1]], add=True)
                return 0

            lax.fori_loop(0, CH // 2, body, 0)
            plsc.subcore_barrier()

            for off, ln in ZCHUNKS:
                pltpu.sync_copy(acc_s.at[pl.ds(base + off, ln)],
                                bounce_v.at[pl.ds(0, ln)])
                pltpu.sync_copy(bounce_v.at[pl.ds(0, ln)],
                                out_hbm.at[c * H + h, pl.ds(base + off, ln)])
            plsc.subcore_barrier()

    return k(*hs_halves, srcp, dstp, zeros_hbm).reshape(NC, H, NPAD, DH)


def _tc_matmul_scale(xp, W, deg1):
    kdim, d = W.shape
    nh = d // DH

    def body(x_ref, w_ref, deg_ref, *o_refs):
        h = jnp.dot(x_ref[...], w_ref[...], preferred_element_type=jnp.float32)
        hs = h * lax.rsqrt(deg_ref[...])
        for q, o_ref in enumerate(o_refs):
            o_ref[...] = hs[:, q * DH:(q + 1) * DH]

    return pl.pallas_call(
        body,
        grid=(NB,),
        in_specs=[
            pl.BlockSpec((BR, kdim), lambda i: (i, 0)),
            pl.BlockSpec((kdim, d), lambda i: (0, 0)),
            pl.BlockSpec((BR, 1), lambda i: (i, 0)),
        ],
        out_specs=[pl.BlockSpec((BR, DH), lambda i: (i, 0)) for _ in range(nh)],
        out_shape=[jax.ShapeDtypeStruct((NPAD, DH), jnp.float32)
                   for _ in range(nh)],
    )(xp, W, deg1)


def _tc_bn_relu_matmul_scale(z, scale, shift, W, deg1):
    kdim, d = W.shape
    nh = d // DH

    def body(z_ref, sc_ref, sh_ref, w_ref, deg_ref, *o_refs):
        a = jax.nn.relu(z_ref[...] * sc_ref[...] + sh_ref[...])
        h = jnp.dot(a, w_ref[...], preferred_element_type=jnp.float32)
        hs = h * lax.rsqrt(deg_ref[...])
        for q, o_ref in enumerate(o_refs):
            o_ref[...] = hs[:, q * DH:(q + 1) * DH]

    return pl.pallas_call(
        body,
        grid=(NB,),
        in_specs=[
            pl.BlockSpec((BR, kdim), lambda i: (i, 0)),
            pl.BlockSpec((1, kdim), lambda i: (0, 0)),
            pl.BlockSpec((1, kdim), lambda i: (0, 0)),
            pl.BlockSpec((kdim, d), lambda i: (0, 0)),
            pl.BlockSpec((BR, 1), lambda i: (i, 0)),
        ],
        out_specs=[pl.BlockSpec((BR, DH), lambda i: (i, 0)) for _ in range(nh)],
        out_shape=[jax.ShapeDtypeStruct((NPAD, DH), jnp.float32)
                   for _ in range(nh)],
    )(z, scale, shift, W, deg1)


def _tc_z_stats(p, hs_a, hs_b, deg1, b):
    d = 2 * DH

    def body(p_ref, hsa_ref, hsb_ref, deg_ref, b_ref, z_ref, st_ref, acc_ref):
        i = pl.program_id(0)
        agg = jnp.concatenate(
            [p_ref[0, 0] + p_ref[1, 0] + hsa_ref[...],
             p_ref[0, 1] + p_ref[1, 1] + hsb_ref[...]], axis=1)
        z = lax.rsqrt(deg_ref[...]) * agg + b_ref[...]
        rows = i * BR + lax.broadcasted_iota(jnp.int32, (BR, d), 0)
        zm = jnp.where(rows < N, z, 0.0)
        z_ref[...] = zm

        @pl.when(i == 0)
        def _():
            acc_ref[...] = jnp.zeros_like(acc_ref)

        sums = jnp.concatenate(
            [jnp.sum(zm, axis=0, keepdims=True),
             jnp.sum(zm * zm, axis=0, keepdims=True)], axis=0)
        acc_ref[...] += sums

        @pl.when(i == NB - 1)
        def _():
            st_ref[...] = acc_ref[...]

    return pl.pallas_call(
        body,
        grid=(NB,),
        in_specs=[
            pl.BlockSpec((NC, 2, BR, DH), lambda i: (0, 0, i, 0)),
            pl.BlockSpec((BR, DH), lambda i: (i, 0)),
            pl.BlockSpec((BR, DH), lambda i: (i, 0)),
            pl.BlockSpec((BR, 1), lambda i: (i, 0)),
            pl.BlockSpec((1, d), lambda i: (0, 0)),
        ],
        out_specs=[
            pl.BlockSpec((BR, d), lambda i: (i, 0)),
            pl.BlockSpec((2, d), lambda i: (0, 0)),
        ],
        out_shape=[
            jax.ShapeDtypeStruct((NPAD, d), jnp.float32),
            jax.ShapeDtypeStruct((2, d), jnp.float32),
        ],
        scratch_shapes=[pltpu.VMEM((2, d), jnp.float32)],
        compiler_params=pltpu.CompilerParams(dimension_semantics=("arbitrary",)),
    )(p, hs_a, hs_b, deg1, b)


def _tc_final(p, hs, deg1, b):
    d = D_OUTP
    rb = BR

    def body(p_ref, hs_ref, deg_ref, b_ref, o_ref):
        agg = p_ref[0, 0] + p_ref[1, 0] + hs_ref[...]
        z = lax.rsqrt(deg_ref[...]) * agg + b_ref[...]
        cols = lax.broadcasted_iota(jnp.int32, (rb, d), 1)
        zm = jnp.where(cols < D_OUT, z, -jnp.inf)
        m = jnp.max(zm, axis=1, keepdims=True)
        lse = jnp.log(jnp.sum(jnp.exp(zm - m), axis=1, keepdims=True))
        o_ref[...] = (z - m - lse)[:, :D_OUT]

    return pl.pallas_call(
        body,
        grid=(NB,),
        in_specs=[
            pl.BlockSpec((NC, 1, rb, d), lambda i: (0, 0, i, 0)),
            pl.BlockSpec((rb, d), lambda i: (i, 0)),
            pl.BlockSpec((rb, 1), lambda i: (i, 0)),
            pl.BlockSpec((1, d), lambda i: (0, 0)),
        ],
        out_specs=pl.BlockSpec((rb, D_OUT), lambda i: (i, 0)),
        out_shape=jax.ShapeDtypeStruct((N, D_OUT), jnp.float32),
    )(p, hs, deg1, b)


def kernel(x, edge_index, W1, b1, gamma1, beta1, W2, b2, gamma2, beta2, W3, b3):
    src = edge_index[0]
    dst = edge_index[1]
    padlen = ETOT - E
    fill = jnp.full((padlen,), N, jnp.int32)
    srcp = jnp.concatenate([src, fill]).reshape(NW, CH, CHUNK)
    dstp = jnp.concatenate([dst, fill]).reshape(NW, CH, CHUNK)
    zeros64 = jnp.zeros((ZCH, DH), jnp.float32)
    zeros1d = jnp.zeros((NPAD,), jnp.float32)

    degp = _sc_degree(dstp, zeros1d)
    deg1 = _tc_degsum(degp).reshape(NPAD, 1)

    xp = jnp.pad(x, ((0, NPAD - N), (0, 0)))

    hs1a, hs1b = _tc_matmul_scale(xp, W1, deg1)
    p1 = _sc_agg((hs1a, hs1b), srcp, dstp, zeros64)
    z1, st1 = _tc_z_stats(p1, hs1a, hs1b, deg1, b1.reshape(1, -1))
    mean1 = st1[0] / N
    var1 = st1[1] / N - mean1 * mean1
    isd1 = gamma1 * lax.rsqrt(var1 + EPS)
    sc1 = isd1.reshape(1, -1)
    sh1 = (beta1 - mean1 * isd1).reshape(1, -1)

    hs2a, hs2b = _tc_bn_relu_matmul_scale(z1, sc1, sh1, W2, deg1)
    p2 = _sc_agg((hs2a, hs2b), srcp, dstp, zeros64)
    z2, st2 = _tc_z_stats(p2, hs2a, hs2b, deg1, b2.reshape(1, -1))
    mean2 = st2[0] / N
    var2 = st2[1] / N - mean2 * mean2
    isd2 = gamma2 * lax.rsqrt(var2 + EPS)
    sc2 = isd2.reshape(1, -1)
    sh2 = (beta2 - mean2 * isd2).reshape(1, -1)

    W3p = jnp.pad(W3, ((0, 0), (0, D_OUTP - D_OUT)))
    b3p = jnp.pad(b3, (0, D_OUTP - D_OUT)).reshape(1, -1)
    hs3, = _tc_bn_relu_matmul_scale(z2, sc2, sh2, W3p, deg1)
    p3 = _sc_agg((hs3,), srcp, dstp, zeros64)
    return _tc_final(p3, hs3, deg1, b3p)


# speedup vs baseline: 8.7884x; 1.0832x over previous
import functools

import jax
import jax.numpy as jnp
from jax import lax
from jax.experimental import pallas as pl
from jax.experimental.pallas import tpu as pltpu
from jax.experimental.pallas import tpu_sc as plsc

N = 10000
E = 320000
D_HID = 128
D_OUT = 40
D_OUTP = 64
EPS = 1e-5

NC = 2
NS = 16
NW = NC * NS
CHUNK = 128
CH = 80
EPT = CH * CHUNK
ETOT = NW * EPT
NPAD = 10112
SR = 79
DH = 64
NB = 16
BR = NPAD // NB
STRIPE = NPAD // NS
NBUF = 4


def _sc_mesh():
    return plsc.VectorSubcoreMesh(
        core_axis_name="c", subcore_axis_name="s", num_cores=NC, num_subcores=NS
    )


def _sc_degree(dstp, zeros_hbm):
    @functools.partial(
        pl.kernel,
        out_type=jax.ShapeDtypeStruct((NW, NPAD), jnp.float32),
        mesh=_sc_mesh(),
        scratch_types=[
            pltpu.VMEM((CH, CHUNK), jnp.int32),
            pltpu.VMEM((NPAD,), jnp.float32),
        ],
        compiler_params=pltpu.CompilerParams(needs_layout_passes=False),
    )
    def k(dst_hbm, zero_hbm, out_hbm, dst_v, hist_v):
        c = lax.axis_index("c")
        s = lax.axis_index("s")
        wid = c * NS + s
        pltpu.sync_copy(dst_hbm.at[wid], dst_v)
        pltpu.sync_copy(zero_hbm, hist_v)

        ones = jnp.full((16,), 1.0, jnp.float32)

        def body(j, _):
            for kk in range(CHUNK // 16):
                idx = dst_v[j, pl.ds(kk * 16, 16)]
                plsc.addupdate_scatter(hist_v, [idx], ones)
            return 0

        lax.fori_loop(0, CH, body, 0)
        pltpu.sync_copy(hist_v, out_hbm.at[wid])

    return k(dstp, zeros_hbm)


def _tc_degsum(degp):
    def body(p_ref, o_ref):
        o_ref[...] = jnp.sum(p_ref[...], axis=0, keepdims=True) + 1.0

    return pl.pallas_call(
        body,
        out_shape=jax.ShapeDtypeStruct((1, NPAD), jnp.float32),
    )(degp)


def _sc_agg(hs_halves, srcp, dstp, zeros_hbm):
    H = len(hs_halves)

    @functools.partial(
        pl.kernel,
        out_type=jax.ShapeDtypeStruct((NC * H, NPAD, DH), jnp.float32),
        mesh=_sc_mesh(),
        scratch_types=[
            pltpu.VMEM((CH, CHUNK), jnp.int32),
            pltpu.VMEM((CH, CHUNK), jnp.int32),
            [pltpu.VMEM((CHUNK, DH), jnp.float32) for _ in range(NBUF)],
            [pltpu.SemaphoreType.DMA for _ in range(NBUF)],
            [pltpu.SemaphoreType.DMA for _ in range(NBUF)],
            pltpu.VMEM_SHARED((NPAD, DH), jnp.float32),
        ],
        compiler_params=pltpu.CompilerParams(use_tc_tiling_on_sc=False),
    )
    def k(*refs):
        hs_hbms = refs[:H]
        (src_hbm, dst_hbm, zero_hbm, out_hbm,
         src_v, dst_v, rows, gsem, ssem, acc_s) = refs[H:]
        c = lax.axis_index("c")
        s = lax.axis_index("s")
        wid = c * NS + s
        pltpu.sync_copy(src_hbm.at[wid], src_v)
        pltpu.sync_copy(dst_hbm.at[wid], dst_v)
        base = s * STRIPE

        for h in range(H):
            hs_hbm = hs_hbms[h]
            pltpu.sync_copy(zero_hbm, acc_s.at[pl.ds(base, STRIPE)])
            plsc.subcore_barrier()

            for q in range(NBUF):
                pltpu.async_copy(hs_hbm.at[src_v.at[q]], rows[q], gsem[q])

            def body(g, _, hs_hbm=hs_hbm):
                j = g * NBUF
                for q in range(NBUF):
                    pltpu.make_async_copy(
                        hs_hbm.at[src_v.at[j + q]], rows[q], gsem[q]).wait()
                    pltpu.async_copy(
                        rows[q], acc_s.at[dst_v.at[j + q]], ssem[q], add=True)
                for q in range(NBUF):
                    pltpu.make_async_copy(
                        rows[q], acc_s.at[dst_v.at[j + q]], ssem[q]).wait()

                    @pl.when(j + NBUF + q < CH)
                    def _(q=q):
                        pltpu.async_copy(
                            hs_hbm.at[src_v.at[j + NBUF + q]], rows[q], gsem[q])
                return 0

            lax.fori_loop(0, CH // NBUF, body, 0)
            plsc.subcore_barrier()

            pltpu.sync_copy(acc_s.at[pl.ds(base, STRIPE)],
                            out_hbm.at[c * H + h, pl.ds(base, STRIPE)])
            plsc.subcore_barrier()

    return k(*hs_halves, srcp, dstp, zeros_hbm)


def _tc_matmul_scale(xp, W, deg1):
    kdim, d = W.shape
    nh = d // DH

    def body(x_ref, w_ref, deg_ref, *o_refs):
        h = jnp.dot(x_ref[...], w_ref[...], preferred_element_type=jnp.float32)
        hs = h * lax.rsqrt(deg_ref[...])
        for q, o_ref in enumerate(o_refs):
            o_ref[...] = hs[:, q * DH:(q + 1) * DH]

    return pl.pallas_call(
        body,
        grid=(NB,),
        in_specs=[
            pl.BlockSpec((BR, kdim), lambda i: (i, 0)),
            pl.BlockSpec((kdim, d), lambda i: (0, 0)),
            pl.BlockSpec((BR, 1), lambda i: (i, 0)),
        ],
        out_specs=[pl.BlockSpec((BR, DH), lambda i: (i, 0)) for _ in range(nh)],
        out_shape=[jax.ShapeDtypeStruct((NPAD, DH), jnp.float32)
                   for _ in range(nh)],
    )(xp, W, deg1)


def _tc_bn_relu_matmul_scale(z, scale, shift, W, deg1):
    kdim, d = W.shape
    nh = d // DH

    def body(z_ref, sc_ref, sh_ref, w_ref, deg_ref, *o_refs):
        a = jax.nn.relu(z_ref[...] * sc_ref[...] + sh_ref[...])
        h = jnp.dot(a, w_ref[...], preferred_element_type=jnp.float32)
        hs = h * lax.rsqrt(deg_ref[...])
        for q, o_ref in enumerate(o_refs):
            o_ref[...] = hs[:, q * DH:(q + 1) * DH]

    return pl.pallas_call(
        body,
        grid=(NB,),
        in_specs=[
            pl.BlockSpec((BR, kdim), lambda i: (i, 0)),
            pl.BlockSpec((1, kdim), lambda i: (0, 0)),
            pl.BlockSpec((1, kdim), lambda i: (0, 0)),
            pl.BlockSpec((kdim, d), lambda i: (0, 0)),
            pl.BlockSpec((BR, 1), lambda i: (i, 0)),
        ],
        out_specs=[pl.BlockSpec((BR, DH), lambda i: (i, 0)) for _ in range(nh)],
        out_shape=[jax.ShapeDtypeStruct((NPAD, DH), jnp.float32)
                   for _ in range(nh)],
    )(z, scale, shift, W, deg1)


def _tc_z_stats(p, hs_a, hs_b, deg1, b):
    d = 2 * DH

    def body(p_ref, hsa_ref, hsb_ref, deg_ref, b_ref, z_ref, st_ref, acc_ref):
        i = pl.program_id(0)
        agg = jnp.concatenate(
            [p_ref[0] + p_ref[2] + hsa_ref[...],
             p_ref[1] + p_ref[3] + hsb_ref[...]], axis=1)
        z = lax.rsqrt(deg_ref[...]) * agg + b_ref[...]
        rows = i * BR + lax.broadcasted_iota(jnp.int32, (BR, d), 0)
        zm = jnp.where(rows < N, z, 0.0)
        z_ref[...] = zm

        @pl.when(i == 0)
        def _():
            acc_ref[...] = jnp.zeros_like(acc_ref)

        sums = jnp.concatenate(
            [jnp.sum(zm, axis=0, keepdims=True),
             jnp.sum(zm * zm, axis=0, keepdims=True)], axis=0)
        acc_ref[...] += sums

        @pl.when(i == NB - 1)
        def _():
            st_ref[...] = acc_ref[...]

    return pl.pallas_call(
        body,
        grid=(NB,),
        in_specs=[
            pl.BlockSpec((NC * 2, BR, DH), lambda i: (0, i, 0)),
            pl.BlockSpec((BR, DH), lambda i: (i, 0)),
            pl.BlockSpec((BR, DH), lambda i: (i, 0)),
            pl.BlockSpec((BR, 1), lambda i: (i, 0)),
            pl.BlockSpec((1, d), lambda i: (0, 0)),
        ],
        out_specs=[
            pl.BlockSpec((BR, d), lambda i: (i, 0)),
            pl.BlockSpec((2, d), lambda i: (0, 0)),
        ],
        out_shape=[
            jax.ShapeDtypeStruct((NPAD, d), jnp.float32),
            jax.ShapeDtypeStruct((2, d), jnp.float32),
        ],
        scratch_shapes=[pltpu.VMEM((2, d), jnp.float32)],
        compiler_params=pltpu.CompilerParams(dimension_semantics=("arbitrary",)),
    )(p, hs_a, hs_b, deg1, b)


def _tc_final(p, hs, deg1, b):
    d = D_OUTP
    rb = BR

    def body(p_ref, hs_ref, deg_ref, b_ref, o_ref):
        agg = p_ref[0] + p_ref[1] + hs_ref[...]
        z = lax.rsqrt(deg_ref[...]) * agg + b_ref[...]
        cols = lax.broadcasted_iota(jnp.int32, (rb, d), 1)
        zm = jnp.where(cols < D_OUT, z, -jnp.inf)
        m = jnp.max(zm, axis=1, keepdims=True)
        lse = jnp.log(jnp.sum(jnp.exp(zm - m), axis=1, keepdims=True))
        o_ref[...] = (z - m - lse)[:, :D_OUT]

    return pl.pallas_call(
        body,
        grid=(NB,),
        in_specs=[
            pl.BlockSpec((NC, rb, d), lambda i: (0, i, 0)),
            pl.BlockSpec((rb, d), lambda i: (i, 0)),
            pl.BlockSpec((rb, 1), lambda i: (i, 0)),
            pl.BlockSpec((1, d), lambda i: (0, 0)),
        ],
        out_specs=pl.BlockSpec((rb, D_OUT), lambda i: (i, 0)),
        out_shape=jax.ShapeDtypeStruct((N, D_OUT), jnp.float32),
    )(p, hs, deg1, b)


def kernel(x, edge_index, W1, b1, gamma1, beta1, W2, b2, gamma2, beta2, W3, b3):
    src = edge_index[0]
    dst = edge_index[1]
    padlen = ETOT - E
    fill = jnp.full((padlen,), N, jnp.int32)
    srcp = jnp.concatenate([src, fill]).reshape(NW, CH, CHUNK)
    dstp = jnp.concatenate([dst, fill]).reshape(NW, CH, CHUNK)
    zeros64 = jnp.zeros((STRIPE, DH), jnp.float32)
    zeros1d = jnp.zeros((NPAD,), jnp.float32)

    degp = _sc_degree(dstp, zeros1d)
    deg1 = _tc_degsum(degp).reshape(NPAD, 1)

    xp = jnp.pad(x, ((0, NPAD - N), (0, 0)))

    hs1a, hs1b = _tc_matmul_scale(xp, W1, deg1)
    p1 = _sc_agg((hs1a, hs1b), srcp, dstp, zeros64)
    z1, st1 = _tc_z_stats(p1, hs1a, hs1b, deg1, b1.reshape(1, -1))
    mean1 = st1[0] / N
    var1 = st1[1] / N - mean1 * mean1
    isd1 = gamma1 * lax.rsqrt(var1 + EPS)
    sc1 = isd1.reshape(1, -1)
    sh1 = (beta1 - mean1 * isd1).reshape(1, -1)

    hs2a, hs2b = _tc_bn_relu_matmul_scale(z1, sc1, sh1, W2, deg1)
    p2 = _sc_agg((hs2a, hs2b), srcp, dstp, zeros64)
    z2, st2 = _tc_z_stats(p2, hs2a, hs2b, deg1, b2.reshape(1, -1))
    mean2 = st2[0] / N
    var2 = st2[1] / N - mean2 * mean2
    isd2 = gamma2 * lax.rsqrt(var2 + EPS)
    sc2 = isd2.reshape(1, -1)
    sh2 = (beta2 - mean2 * isd2).reshape(1, -1)

    W3p = jnp.pad(W3, ((0, 0), (0, D_OUTP - D_OUT)))
    b3p = jnp.pad(b3, (0, D_OUTP - D_OUT)).reshape(1, -1)
    hs3, = _tc_bn_relu_matmul_scale(z2, sc2, sh2, W3p, deg1)
    p3 = _sc_agg((hs3,), srcp, dstp, zeros64)
    return _tc_final(p3, hs3, deg1, b3p)


# speedup vs baseline: 16.8565x; 1.9180x over previous
import functools

import jax
import jax.numpy as jnp
from jax import lax
from jax.experimental import pallas as pl
from jax.experimental.pallas import tpu as pltpu
from jax.experimental.pallas import tpu_sc as plsc

N = 10000
E = 320000
D_HID = 128
D_OUT = 40
D_OUTP = 64
EPS = 1e-5

NC = 2
NS = 16
NW = NC * NS
CHUNK = 128
CH = 80
EPT = CH * CHUNK
ETOT = NW * EPT
NPAD = 10112
SR = 79
DH = 64
NB = 16
BR = NPAD // NB
STRIPE = NPAD // NS
NBUF = 3


def _sc_mesh():
    return plsc.VectorSubcoreMesh(
        core_axis_name="c", subcore_axis_name="s", num_cores=NC, num_subcores=NS
    )


def _sc_degree(dstp, zeros_hbm):
    @functools.partial(
        pl.kernel,
        out_type=jax.ShapeDtypeStruct((NW, NPAD), jnp.float32),
        mesh=_sc_mesh(),
        scratch_types=[
            pltpu.VMEM((CH, CHUNK), jnp.int32),
            pltpu.VMEM((NPAD,), jnp.float32),
        ],
        compiler_params=pltpu.CompilerParams(needs_layout_passes=False),
    )
    def k(dst_hbm, zero_hbm, out_hbm, dst_v, hist_v):
        c = lax.axis_index("c")
        s = lax.axis_index("s")
        wid = c * NS + s
        pltpu.sync_copy(dst_hbm.at[wid], dst_v)
        pltpu.sync_copy(zero_hbm, hist_v)

        ones = jnp.full((16,), 1.0, jnp.float32)

        def body(j, _):
            for kk in range(CHUNK // 16):
                idx = dst_v[j, pl.ds(kk * 16, 16)]
                plsc.addupdate_scatter(hist_v, [idx], ones)
            return 0

        lax.fori_loop(0, CH, body, 0)
        pltpu.sync_copy(hist_v, out_hbm.at[wid])

    return k(dstp, zeros_hbm)


def _tc_degsum(degp):
    def body(p_ref, o_ref):
        o_ref[...] = jnp.sum(p_ref[...], axis=0, keepdims=True) + 1.0

    return pl.pallas_call(
        body,
        out_shape=jax.ShapeDtypeStruct((1, NPAD), jnp.float32),
    )(degp)


def _sc_agg(hs_halves, srcp, dstp, zeros_hbm):
    H = len(hs_halves)

    @functools.partial(
        pl.kernel,
        out_type=jax.ShapeDtypeStruct((NC * H, NPAD, DH), jnp.float32),
        mesh=_sc_mesh(),
        scratch_types=[
            pltpu.VMEM((CH, CHUNK), jnp.int32),
            pltpu.VMEM((CH, CHUNK), jnp.int32),
            [pltpu.VMEM((CHUNK, DH), jnp.float32) for _ in range(NBUF)],
            [pltpu.SemaphoreType.DMA for _ in range(NBUF)],
            [pltpu.SemaphoreType.DMA for _ in range(NBUF)],
            pltpu.VMEM_SHARED((NPAD, DH), jnp.float32),
            pltpu.VMEM_SHARED((NPAD, DH), jnp.float32),
        ],
        compiler_params=pltpu.CompilerParams(use_tc_tiling_on_sc=False),
    )
    def k(*refs):
        hs_hbms = refs[:H]
        (src_hbm, dst_hbm, zero_hbm, out_hbm,
         src_v, dst_v, rows, gsem, ssem, acc_s, hs_s) = refs[H:]
        c = lax.axis_index("c")
        s = lax.axis_index("s")
        wid = c * NS + s
        pltpu.sync_copy(src_hbm.at[wid], src_v)
        pltpu.sync_copy(dst_hbm.at[wid], dst_v)
        base = s * STRIPE

        for h in range(H):
            hs_hbm = hs_hbms[h]
            pltpu.sync_copy(zero_hbm, acc_s.at[pl.ds(base, STRIPE)])
            pltpu.sync_copy(hs_hbm.at[pl.ds(base, STRIPE)],
                            hs_s.at[pl.ds(base, STRIPE)])
            plsc.subcore_barrier()

            for q in range(NBUF):
                pltpu.async_copy(hs_s.at[src_v.at[q]], rows[q], gsem[q])

            def body(g, _):
                j = g * NBUF
                for q in range(NBUF):
                    @pl.when(j + q < CH)
                    def _(q=q):
                        pltpu.make_async_copy(
                            hs_s.at[src_v.at[j + q]], rows[q], gsem[q]).wait()
                        pltpu.async_copy(
                            rows[q], acc_s.at[dst_v.at[j + q]], ssem[q], add=True)
                for q in range(NBUF):
                    @pl.when(j + q < CH)
                    def _(q=q):
                        pltpu.make_async_copy(
                            rows[q], acc_s.at[dst_v.at[j + q]], ssem[q]).wait()

                    @pl.when(j + NBUF + q < CH)
                    def _(q=q):
                        pltpu.async_copy(
                            hs_s.at[src_v.at[j + NBUF + q]], rows[q], gsem[q])
                return 0

            lax.fori_loop(0, (CH + NBUF - 1) // NBUF, body, 0)
            plsc.subcore_barrier()

            pltpu.sync_copy(acc_s.at[pl.ds(base, STRIPE)],
                            out_hbm.at[c * H + h, pl.ds(base, STRIPE)])
            plsc.subcore_barrier()

    return k(*hs_halves, srcp, dstp, zeros_hbm)


def _tc_matmul_scale(xp, W, deg1):
    kdim, d = W.shape
    nh = d // DH

    def body(x_ref, w_ref, deg_ref, *o_refs):
        h = jnp.dot(x_ref[...], w_ref[...], preferred_element_type=jnp.float32)
        hs = h * lax.rsqrt(deg_ref[...])
        for q, o_ref in enumerate(o_refs):
            o_ref[...] = hs[:, q * DH:(q + 1) * DH]

    return pl.pallas_call(
        body,
        grid=(NB,),
        in_specs=[
            pl.BlockSpec((BR, kdim), lambda i: (i, 0)),
            pl.BlockSpec((kdim, d), lambda i: (0, 0)),
            pl.BlockSpec((BR, 1), lambda i: (i, 0)),
        ],
        out_specs=[pl.BlockSpec((BR, DH), lambda i: (i, 0)) for _ in range(nh)],
        out_shape=[jax.ShapeDtypeStruct((NPAD, DH), jnp.float32)
                   for _ in range(nh)],
    )(xp, W, deg1)


def _tc_bn_relu_matmul_scale(z, scale, shift, W, deg1):
    kdim, d = W.shape
    nh = d // DH

    def body(z_ref, sc_ref, sh_ref, w_ref, deg_ref, *o_refs):
        a = jax.nn.relu(z_ref[...] * sc_ref[...] + sh_ref[...])
        h = jnp.dot(a, w_ref[...], preferred_element_type=jnp.float32)
        hs = h * lax.rsqrt(deg_ref[...])
        for q, o_ref in enumerate(o_refs):
            o_ref[...] = hs[:, q * DH:(q + 1) * DH]

    return pl.pallas_call(
        body,
        grid=(NB,),
        in_specs=[
            pl.BlockSpec((BR, kdim), lambda i: (i, 0)),
            pl.BlockSpec((1, kdim), lambda i: (0, 0)),
            pl.BlockSpec((1, kdim), lambda i: (0, 0)),
            pl.BlockSpec((kdim, d), lambda i: (0, 0)),
            pl.BlockSpec((BR, 1), lambda i: (i, 0)),
        ],
        out_specs=[pl.BlockSpec((BR, DH), lambda i: (i, 0)) for _ in range(nh)],
        out_shape=[jax.ShapeDtypeStruct((NPAD, DH), jnp.float32)
                   for _ in range(nh)],
    )(z, scale, shift, W, deg1)


def _tc_z_stats(p, hs_a, hs_b, deg1, b):
    d = 2 * DH

    def body(p_ref, hsa_ref, hsb_ref, deg_ref, b_ref, z_ref, st_ref, acc_ref):
        i = pl.program_id(0)
        agg = jnp.concatenate(
            [p_ref[0] + p_ref[2] + hsa_ref[...],
             p_ref[1] + p_ref[3] + hsb_ref[...]], axis=1)
        z = lax.rsqrt(deg_ref[...]) * agg + b_ref[...]
        rows = i * BR + lax.broadcasted_iota(jnp.int32, (BR, d), 0)
        zm = jnp.where(rows < N, z, 0.0)
        z_ref[...] = zm

        @pl.when(i == 0)
        def _():
            acc_ref[...] = jnp.zeros_like(acc_ref)

        sums = jnp.concatenate(
            [jnp.sum(zm, axis=0, keepdims=True),
             jnp.sum(zm * zm, axis=0, keepdims=True)], axis=0)
        acc_ref[...] += sums

        @pl.when(i == NB - 1)
        def _():
            st_ref[...] = acc_ref[...]

    return pl.pallas_call(
        body,
        grid=(NB,),
        in_specs=[
            pl.BlockSpec((NC * 2, BR, DH), lambda i: (0, i, 0)),
            pl.BlockSpec((BR, DH), lambda i: (i, 0)),
            pl.BlockSpec((BR, DH), lambda i: (i, 0)),
            pl.BlockSpec((BR, 1), lambda i: (i, 0)),
            pl.BlockSpec((1, d), lambda i: (0, 0)),
        ],
        out_specs=[
            pl.BlockSpec((BR, d), lambda i: (i, 0)),
            pl.BlockSpec((2, d), lambda i: (0, 0)),
        ],
        out_shape=[
            jax.ShapeDtypeStruct((NPAD, d), jnp.float32),
            jax.ShapeDtypeStruct((2, d), jnp.float32),
        ],
        scratch_shapes=[pltpu.VMEM((2, d), jnp.float32)],
        compiler_params=pltpu.CompilerParams(dimension_semantics=("arbitrary",)),
    )(p, hs_a, hs_b, deg1, b)


def _tc_final(p, hs, deg1, b):
    d = D_OUTP
    rb = BR

    def body(p_ref, hs_ref, deg_ref, b_ref, o_ref):
        agg = p_ref[0] + p_ref[1] + hs_ref[...]
        z = lax.rsqrt(deg_ref[...]) * agg + b_ref[...]
        cols = lax.broadcasted_iota(jnp.int32, (rb, d), 1)
        zm = jnp.where(cols < D_OUT, z, -jnp.inf)
        m = jnp.max(zm, axis=1, keepdims=True)
        lse = jnp.log(jnp.sum(jnp.exp(zm - m), axis=1, keepdims=True))
        o_ref[...] = (z - m - lse)[:, :D_OUT]

    return pl.pallas_call(
        body,
        grid=(NB,),
        in_specs=[
            pl.BlockSpec((NC, rb, d), lambda i: (0, i, 0)),
            pl.BlockSpec((rb, d), lambda i: (i, 0)),
            pl.BlockSpec((rb, 1), lambda i: (i, 0)),
            pl.BlockSpec((1, d), lambda i: (0, 0)),
        ],
        out_specs=pl.BlockSpec((rb, D_OUT), lambda i: (i, 0)),
        out_shape=jax.ShapeDtypeStruct((N, D_OUT), jnp.float32),
    )(p, hs, deg1, b)


def kernel(x, edge_index, W1, b1, gamma1, beta1, W2, b2, gamma2, beta2, W3, b3):
    src = edge_index[0]
    dst = edge_index[1]
    padlen = ETOT - E
    fill = jnp.full((padlen,), N, jnp.int32)
    srcp = jnp.concatenate([src, fill]).reshape(NW, CH, CHUNK)
    dstp = jnp.concatenate([dst, fill]).reshape(NW, CH, CHUNK)
    zeros64 = jnp.zeros((STRIPE, DH), jnp.float32)
    zeros1d = jnp.zeros((NPAD,), jnp.float32)

    degp = _sc_degree(dstp, zeros1d)
    deg1 = _tc_degsum(degp).reshape(NPAD, 1)

    xp = jnp.pad(x, ((0, NPAD - N), (0, 0)))

    hs1a, hs1b = _tc_matmul_scale(xp, W1, deg1)
    p1 = _sc_agg((hs1a, hs1b), srcp, dstp, zeros64)
    z1, st1 = _tc_z_stats(p1, hs1a, hs1b, deg1, b1.reshape(1, -1))
    mean1 = st1[0] / N
    var1 = st1[1] / N - mean1 * mean1
    isd1 = gamma1 * lax.rsqrt(var1 + EPS)
    sc1 = isd1.reshape(1, -1)
    sh1 = (beta1 - mean1 * isd1).reshape(1, -1)

    hs2a, hs2b = _tc_bn_relu_matmul_scale(z1, sc1, sh1, W2, deg1)
    p2 = _sc_agg((hs2a, hs2b), srcp, dstp, zeros64)
    z2, st2 = _tc_z_stats(p2, hs2a, hs2b, deg1, b2.reshape(1, -1))
    mean2 = st2[0] / N
    var2 = st2[1] / N - mean2 * mean2
    isd2 = gamma2 * lax.rsqrt(var2 + EPS)
    sc2 = isd2.reshape(1, -1)
    sh2 = (beta2 - mean2 * isd2).reshape(1, -1)

    W3p = jnp.pad(W3, ((0, 0), (0, D_OUTP - D_OUT)))
    b3p = jnp.pad(b3, (0, D_OUTP - D_OUT)).reshape(1, -1)
    hs3, = _tc_bn_relu_matmul_scale(z2, sc2, sh2, W3p, deg1)
    p3 = _sc_agg((hs3,), srcp, dstp, zeros64)
    return _tc_final(p3, hs3, deg1, b3p)


# speedup vs baseline: 17.4523x; 1.0353x over previous
import functools

import jax
import jax.numpy as jnp
from jax import lax
from jax.experimental import pallas as pl
from jax.experimental.pallas import tpu as pltpu
from jax.experimental.pallas import tpu_sc as plsc

N = 10000
E = 320000
D_HID = 128
D_OUT = 40
D_OUTP = 48
EPS = 1e-5

NC = 2
NS = 16
NW = NC * NS
CHUNK = 128
CH = 80
EPT = CH * CHUNK
ETOT = NW * EPT
NPAD = 10112
SR = 79
DH = 64
NB = 16
BR = NPAD // NB
STRIPE = NPAD // NS
NBUF = 3


def _sc_mesh():
    return plsc.VectorSubcoreMesh(
        core_axis_name="c", subcore_axis_name="s", num_cores=NC, num_subcores=NS
    )


def _sc_degree(dstp, zeros_hbm):
    @functools.partial(
        pl.kernel,
        out_type=jax.ShapeDtypeStruct((NW, NPAD), jnp.float32),
        mesh=_sc_mesh(),
        scratch_types=[
            pltpu.VMEM((CH, CHUNK), jnp.int32),
            pltpu.VMEM((NPAD,), jnp.float32),
        ],
        compiler_params=pltpu.CompilerParams(needs_layout_passes=False),
    )
    def k(dst_hbm, zero_hbm, out_hbm, dst_v, hist_v):
        c = lax.axis_index("c")
        s = lax.axis_index("s")
        wid = c * NS + s
        pltpu.sync_copy(dst_hbm.at[wid], dst_v)
        pltpu.sync_copy(zero_hbm, hist_v)

        ones = jnp.full((16,), 1.0, jnp.float32)

        def body(j, _):
            for kk in range(CHUNK // 16):
                idx = dst_v[j, pl.ds(kk * 16, 16)]
                plsc.addupdate_scatter(hist_v, [idx], ones)
            return 0

        lax.fori_loop(0, CH, body, 0)
        pltpu.sync_copy(hist_v, out_hbm.at[wid])

    return k(dstp, zeros_hbm)


def _tc_degsum(degp):
    def body(p_ref, o_ref):
        o_ref[...] = jnp.sum(p_ref[...], axis=0, keepdims=True) + 1.0

    return pl.pallas_call(
        body,
        out_shape=jax.ShapeDtypeStruct((1, NPAD), jnp.float32),
    )(degp)


def _sc_agg(hs_halves, srcp, dstp, zeros_hbm, d=DH):
    H = len(hs_halves)

    @functools.partial(
        pl.kernel,
        out_type=jax.ShapeDtypeStruct((NC * H, NPAD, d), jnp.float32),
        mesh=_sc_mesh(),
        scratch_types=[
            pltpu.VMEM((CH, CHUNK), jnp.int32),
            pltpu.VMEM((CH, CHUNK), jnp.int32),
            [pltpu.VMEM((CHUNK, d), jnp.float32) for _ in range(NBUF)],
            [pltpu.SemaphoreType.DMA for _ in range(NBUF)],
            [pltpu.SemaphoreType.DMA for _ in range(NBUF)],
            pltpu.VMEM_SHARED((NPAD, d), jnp.float32),
            pltpu.VMEM_SHARED((NPAD, d), jnp.float32),
        ],
        compiler_params=pltpu.CompilerParams(use_tc_tiling_on_sc=False),
    )
    def k(*refs):
        hs_hbms = refs[:H]
        (src_hbm, dst_hbm, zero_hbm, out_hbm,
         src_v, dst_v, rows, gsem, ssem, acc_s, hs_s) = refs[H:]
        c = lax.axis_index("c")
        s = lax.axis_index("s")
        wid = c * NS + s
        pltpu.sync_copy(src_hbm.at[wid], src_v)
        pltpu.sync_copy(dst_hbm.at[wid], dst_v)
        base = s * STRIPE

        for h in range(H):
            hs_hbm = hs_hbms[h]
            pltpu.sync_copy(zero_hbm, acc_s.at[pl.ds(base, STRIPE)])
            pltpu.sync_copy(hs_hbm.at[pl.ds(base, STRIPE)],
                            hs_s.at[pl.ds(base, STRIPE)])
            plsc.subcore_barrier()

            for q in range(NBUF):
                pltpu.async_copy(hs_s.at[src_v.at[q]], rows[q], gsem[q])

            def body(g, _):
                j = g * NBUF
                for q in range(NBUF):
                    @pl.when(j + q < CH)
                    def _(q=q):
                        pltpu.make_async_copy(
                            hs_s.at[src_v.at[j + q]], rows[q], gsem[q]).wait()
                        pltpu.async_copy(
                            rows[q], acc_s.at[dst_v.at[j + q]], ssem[q], add=True)
                for q in range(NBUF):
                    @pl.when(j + q < CH)
                    def _(q=q):
                        pltpu.make_async_copy(
                            rows[q], acc_s.at[dst_v.at[j + q]], ssem[q]).wait()

                    @pl.when(j + NBUF + q < CH)
                    def _(q=q):
                        pltpu.async_copy(
                            hs_s.at[src_v.at[j + NBUF + q]], rows[q], gsem[q])
                return 0

            lax.fori_loop(0, (CH + NBUF - 1) // NBUF, body, 0)
            plsc.subcore_barrier()

            pltpu.sync_copy(acc_s.at[pl.ds(base, STRIPE)],
                            out_hbm.at[c * H + h, pl.ds(base, STRIPE)])
            plsc.subcore_barrier()

    return k(*hs_halves, srcp, dstp, zeros_hbm)


def _tc_matmul_scale(xp, W, deg1):
    kdim, d = W.shape
    nh = d // DH

    def body(x_ref, w_ref, deg_ref, *o_refs):
        h = jnp.dot(x_ref[...], w_ref[...], preferred_element_type=jnp.float32)
        hs = h * lax.rsqrt(deg_ref[...])
        for q, o_ref in enumerate(o_refs):
            o_ref[...] = hs[:, q * DH:(q + 1) * DH]

    return pl.pallas_call(
        body,
        grid=(NB,),
        in_specs=[
            pl.BlockSpec((BR, kdim), lambda i: (i, 0)),
            pl.BlockSpec((kdim, d), lambda i: (0, 0)),
            pl.BlockSpec((BR, 1), lambda i: (i, 0)),
        ],
        out_specs=[pl.BlockSpec((BR, DH), lambda i: (i, 0)) for _ in range(nh)],
        out_shape=[jax.ShapeDtypeStruct((NPAD, DH), jnp.float32)
                   for _ in range(nh)],
    )(xp, W, deg1)


def _tc_bn_relu_matmul_scale(z, scale, shift, W, deg1):
    kdim, d = W.shape
    nh = max(1, d // DH)
    w = d // nh

    def body(z_ref, sc_ref, sh_ref, w_ref, deg_ref, *o_refs):
        a = jax.nn.relu(z_ref[...] * sc_ref[...] + sh_ref[...])
        h = jnp.dot(a, w_ref[...], preferred_element_type=jnp.float32)
        hs = h * lax.rsqrt(deg_ref[...])
        for q, o_ref in enumerate(o_refs):
            o_ref[...] = hs[:, q * w:(q + 1) * w]

    return pl.pallas_call(
        body,
        grid=(NB,),
        in_specs=[
            pl.BlockSpec((BR, kdim), lambda i: (i, 0)),
            pl.BlockSpec((1, kdim), lambda i: (0, 0)),
            pl.BlockSpec((1, kdim), lambda i: (0, 0)),
            pl.BlockSpec((kdim, d), lambda i: (0, 0)),
            pl.BlockSpec((BR, 1), lambda i: (i, 0)),
        ],
        out_specs=[pl.BlockSpec((BR, w), lambda i: (i, 0)) for _ in range(nh)],
        out_shape=[jax.ShapeDtypeStruct((NPAD, w), jnp.float32)
                   for _ in range(nh)],
    )(z, scale, shift, W, deg1)


def _tc_z_stats(p, hs_a, hs_b, deg1, b):
    d = 2 * DH

    def body(p_ref, hsa_ref, hsb_ref, deg_ref, b_ref, z_ref, st_ref, acc_ref):
        i = pl.program_id(0)
        agg = jnp.concatenate(
            [p_ref[0] + p_ref[2] + hsa_ref[...],
             p_ref[1] + p_ref[3] + hsb_ref[...]], axis=1)
        z = lax.rsqrt(deg_ref[...]) * agg + b_ref[...]
        rows = i * BR + lax.broadcasted_iota(jnp.int32, (BR, d), 0)
        zm = jnp.where(rows < N, z, 0.0)
        z_ref[...] = zm

        @pl.when(i == 0)
        def _():
            acc_ref[...] = jnp.zeros_like(acc_ref)

        sums = jnp.concatenate(
            [jnp.sum(zm, axis=0, keepdims=True),
             jnp.sum(zm * zm, axis=0, keepdims=True)], axis=0)
        acc_ref[...] += sums

        @pl.when(i == NB - 1)
        def _():
            st_ref[...] = acc_ref[...]

    return pl.pallas_call(
        body,
        grid=(NB,),
        in_specs=[
            pl.BlockSpec((NC * 2, BR, DH), lambda i: (0, i, 0)),
            pl.BlockSpec((BR, DH), lambda i: (i, 0)),
            pl.BlockSpec((BR, DH), lambda i: (i, 0)),
            pl.BlockSpec((BR, 1), lambda i: (i, 0)),
            pl.BlockSpec((1, d), lambda i: (0, 0)),
        ],
        out_specs=[
            pl.BlockSpec((BR, d), lambda i: (i, 0)),
            pl.BlockSpec((2, d), lambda i: (0, 0)),
        ],
        out_shape=[
            jax.ShapeDtypeStruct((NPAD, d), jnp.float32),
            jax.ShapeDtypeStruct((2, d), jnp.float32),
        ],
        scratch_shapes=[pltpu.VMEM((2, d), jnp.float32)],
        compiler_params=pltpu.CompilerParams(dimension_semantics=("arbitrary",)),
    )(p, hs_a, hs_b, deg1, b)


def _tc_final(p, hs, deg1, b):
    d = D_OUTP
    rb = BR

    def body(p_ref, hs_ref, deg_ref, b_ref, o_ref):
        agg = p_ref[0] + p_ref[1] + hs_ref[...]
        z = lax.rsqrt(deg_ref[...]) * agg + b_ref[...]
        cols = lax.broadcasted_iota(jnp.int32, (rb, d), 1)
        zm = jnp.where(cols < D_OUT, z, -jnp.inf)
        m = jnp.max(zm, axis=1, keepdims=True)
        lse = jnp.log(jnp.sum(jnp.exp(zm - m), axis=1, keepdims=True))
        o_ref[...] = (z - m - lse)[:, :D_OUT]

    return pl.pallas_call(
        body,
        grid=(NB,),
        in_specs=[
            pl.BlockSpec((NC, rb, d), lambda i: (0, i, 0)),
            pl.BlockSpec((rb, d), lambda i: (i, 0)),
            pl.BlockSpec((rb, 1), lambda i: (i, 0)),
            pl.BlockSpec((1, d), lambda i: (0, 0)),
        ],
        out_specs=pl.BlockSpec((rb, D_OUT), lambda i: (i, 0)),
        out_shape=jax.ShapeDtypeStruct((N, D_OUT), jnp.float32),
    )(p, hs, deg1, b)


def kernel(x, edge_index, W1, b1, gamma1, beta1, W2, b2, gamma2, beta2, W3, b3):
    src = edge_index[0]
    dst = edge_index[1]
    padlen = ETOT - E
    fill = jnp.full((padlen,), N, jnp.int32)
    srcp = jnp.concatenate([src, fill]).reshape(NW, CH, CHUNK)
    dstp = jnp.concatenate([dst, fill]).reshape(NW, CH, CHUNK)
    zeros64 = jnp.zeros((STRIPE, DH), jnp.float32)
    zeros48 = jnp.zeros((STRIPE, D_OUTP), jnp.float32)
    zeros1d = jnp.zeros((NPAD,), jnp.float32)

    degp = _sc_degree(dstp, zeros1d)
    deg1 = _tc_degsum(degp).reshape(NPAD, 1)

    xp = jnp.pad(x, ((0, NPAD - N), (0, 0)))

    hs1a, hs1b = _tc_matmul_scale(xp, W1, deg1)
    p1 = _sc_agg((hs1a, hs1b), srcp, dstp, zeros64)
    z1, st1 = _tc_z_stats(p1, hs1a, hs1b, deg1, b1.reshape(1, -1))
    mean1 = st1[0] / N
    var1 = st1[1] / N - mean1 * mean1
    isd1 = gamma1 * lax.rsqrt(var1 + EPS)
    sc1 = isd1.reshape(1, -1)
    sh1 = (beta1 - mean1 * isd1).reshape(1, -1)

    hs2a, hs2b = _tc_bn_relu_matmul_scale(z1, sc1, sh1, W2, deg1)
    p2 = _sc_agg((hs2a, hs2b), srcp, dstp, zeros64)
    z2, st2 = _tc_z_stats(p2, hs2a, hs2b, deg1, b2.reshape(1, -1))
    mean2 = st2[0] / N
    var2 = st2[1] / N - mean2 * mean2
    isd2 = gamma2 * lax.rsqrt(var2 + EPS)
    sc2 = isd2.reshape(1, -1)
    sh2 = (beta2 - mean2 * isd2).reshape(1, -1)

    W3p = jnp.pad(W3, ((0, 0), (0, D_OUTP - D_OUT)))
    b3p = jnp.pad(b3, (0, D_OUTP - D_OUT)).reshape(1, -1)
    hs3, = _tc_bn_relu_matmul_scale(z2, sc2, sh2, W3p, deg1)
    p3 = _sc_agg((hs3,), srcp, dstp, zeros48, d=D_OUTP)
    return _tc_final(p3, hs3, deg1, b3p)
